# Initial kernel scaffold; baseline (speedup 1.0000x reference)
import jax
import jax.numpy as jnp
from jax.experimental import pallas as pl


def kernel(x):
    raise NotImplementedError("write your pallas kernel here")


# speedup vs baseline: 6205.9619x; 6205.9619x over previous
import math

import jax
import jax.numpy as jnp
import numpy as np
from jax.experimental import pallas as pl
from jax.experimental.pallas import tpu as pltpu

_DIM = 128
_BITS = 4
_N_LEVELS = 2 ** _BITS


def _lm_levels(bits: int, n_iter: int = 200) -> np.ndarray:
    n = 2 ** bits
    xs = np.linspace(-10.0, 10.0, 400001)
    pdf = np.exp(-np.abs(xs) * math.sqrt(2.0))
    levels = np.linspace(-3.0, 3.0, n)
    for _ in range(n_iter):
        b = (levels[:-1] + levels[1:]) / 2.0
        idx = np.searchsorted(b, xs)
        num = np.bincount(idx, weights=pdf * xs, minlength=n)
        den = np.bincount(idx, weights=pdf, minlength=n)
        levels = np.where(den > 1e-12, num / np.maximum(den, 1e-12), levels)
    return np.sort(levels).astype(np.float32)


_LEVELS = _lm_levels(_BITS)
_BOUNDS = ((_LEVELS[:-1] + _LEVELS[1:]) / 2.0).astype(np.float32)

_BLOCK_ROWS = 2048


def _body(x_ref, o_ref):
    xb = x_ref[...]
    vmin = jnp.min(xb, axis=1, keepdims=True)
    xc = xb - vmin
    ssq = jnp.sum(xc * xc, axis=1, keepdims=True)
    vscale = jnp.sqrt(ssq) * (1.0 / math.sqrt(_DIM))
    xu = xc / (vscale + 1e-10)
    recon = jnp.full_like(xb, _LEVELS[0])
    for b in range(_N_LEVELS - 1):
        recon = jnp.where(xu > _BOUNDS[b], _LEVELS[b + 1], recon)
    num = jnp.sum(xc * recon, axis=1, keepdims=True)
    den = jnp.sum(recon * recon, axis=1, keepdims=True) + 1e-10
    gamma = num / den
    o_ref[...] = recon * gamma + vmin


def kernel(x):
    shape = x.shape
    rows = math.prod(shape[:-1])
    x2 = x.reshape(rows, _DIM)
    grid = rows // _BLOCK_ROWS
    out = pl.pallas_call(
        _body,
        grid=(grid,),
        in_specs=[pl.BlockSpec((_BLOCK_ROWS, _DIM), lambda i: (i, 0))],
        out_specs=pl.BlockSpec((_BLOCK_ROWS, _DIM), lambda i: (i, 0)),
        out_shape=jax.ShapeDtypeStruct((rows, _DIM), jnp.float32),
    )(x2)
    return out.reshape(shape)


# speedup vs baseline: 6704.5752x; 1.0803x over previous
import math

import jax
import jax.numpy as jnp
import numpy as np
from jax.experimental import pallas as pl
from jax.experimental.pallas import tpu as pltpu

_DIM = 128
_BITS = 4
_N_LEVELS = 2 ** _BITS


def _lm_levels(bits: int, n_iter: int = 200) -> np.ndarray:
    n = 2 ** bits
    xs = np.linspace(-10.0, 10.0, 400001)
    pdf = np.exp(-np.abs(xs) * math.sqrt(2.0))
    levels = np.linspace(-3.0, 3.0, n)
    for _ in range(n_iter):
        b = (levels[:-1] + levels[1:]) / 2.0
        idx = np.searchsorted(b, xs)
        num = np.bincount(idx, weights=pdf * xs, minlength=n)
        den = np.bincount(idx, weights=pdf, minlength=n)
        levels = np.where(den > 1e-12, num / np.maximum(den, 1e-12), levels)
    return np.sort(levels).astype(np.float32)


_LEVELS = _lm_levels(_BITS)
_BOUNDS = ((_LEVELS[:-1] + _LEVELS[1:]) / 2.0).astype(np.float32)

_BLOCK_ROWS = 4096


def _body(x_ref, o_ref):
    xb = x_ref[...]
    vmin = jnp.min(xb, axis=1, keepdims=True)
    xc = xb - vmin
    ssq = jnp.sum(xc * xc, axis=1, keepdims=True)
    rinv = jnp.where(ssq > 0.0, jax.lax.rsqrt(ssq) * math.sqrt(_DIM), 0.0)
    xu = xc * rinv
    recon = jnp.full_like(xb, _LEVELS[0])
    for b in range(_N_LEVELS - 1):
        recon = jnp.where(xu > _BOUNDS[b], _LEVELS[b + 1], recon)
    num = jnp.sum(xc * recon, axis=1, keepdims=True)
    den = jnp.sum(recon * recon, axis=1, keepdims=True) + 1e-10
    gamma = num / den
    o_ref[...] = recon * gamma + vmin


def kernel(x):
    shape = x.shape
    rows = math.prod(shape[:-1])
    x2 = x.reshape(rows, _DIM)
    grid = rows // _BLOCK_ROWS
    out = pl.pallas_call(
        _body,
        grid=(grid,),
        in_specs=[pl.BlockSpec((_BLOCK_ROWS, _DIM), lambda i: (i, 0))],
        out_specs=pl.BlockSpec((_BLOCK_ROWS, _DIM), lambda i: (i, 0)),
        out_shape=jax.ShapeDtypeStruct((rows, _DIM), jnp.float32),
    )(x2)
    return out.reshape(shape)


# speedup vs baseline: 9184.5163x; 1.3699x over previous
import math

import jax
import jax.numpy as jnp
import numpy as np
from jax.experimental import pallas as pl
from jax.experimental.pallas import tpu as pltpu

_DIM = 128
_BITS = 4
_N_LEVELS = 2 ** _BITS


def _lm_levels(bits: int, n_iter: int = 200) -> np.ndarray:
    n = 2 ** bits
    xs = np.linspace(-10.0, 10.0, 400001)
    pdf = np.exp(-np.abs(xs) * math.sqrt(2.0))
    levels = np.linspace(-3.0, 3.0, n)
    for _ in range(n_iter):
        b = (levels[:-1] + levels[1:]) / 2.0
        idx = np.searchsorted(b, xs)
        num = np.bincount(idx, weights=pdf * xs, minlength=n)
        den = np.bincount(idx, weights=pdf, minlength=n)
        levels = np.where(den > 1e-12, num / np.maximum(den, 1e-12), levels)
    return np.sort(levels).astype(np.float32)


_LEVELS = _lm_levels(_BITS)
_BOUNDS = ((_LEVELS[:-1] + _LEVELS[1:]) / 2.0).astype(np.float32)

_BLOCK_ROWS = 4096


def _body(x_ref, o_ref):
    xb = x_ref[...]
    vmin = jnp.min(xb, axis=1, keepdims=True)
    xc = xb - vmin
    ssq = jnp.sum(xc * xc, axis=1, keepdims=True)
    rinv = jnp.where(ssq > 0.0, jax.lax.rsqrt(ssq) * math.sqrt(_DIM), 0.0)
    xu = xc * rinv
    recon = jnp.full_like(xb, _LEVELS[7])
    for b in range(7, _N_LEVELS - 1):
        recon = jnp.where(xu > _BOUNDS[b], _LEVELS[b + 1], recon)
    num = jnp.sum(xc * recon, axis=1, keepdims=True)
    den = jnp.sum(recon * recon, axis=1, keepdims=True) + 1e-10
    gamma = num / den
    o_ref[...] = recon * gamma + vmin


def kernel(x):
    shape = x.shape
    rows = math.prod(shape[:-1])
    x2 = x.reshape(rows, _DIM)
    grid = rows // _BLOCK_ROWS
    out = pl.pallas_call(
        _body,
        grid=(grid,),
        in_specs=[pl.BlockSpec((_BLOCK_ROWS, _DIM), lambda i: (i, 0))],
        out_specs=pl.BlockSpec((_BLOCK_ROWS, _DIM), lambda i: (i, 0)),
        out_shape=jax.ShapeDtypeStruct((rows, _DIM), jnp.float32),
    )(x2)
    return out.reshape(shape)


# speedup vs baseline: 9345.9786x; 1.0176x over previous
import math

import jax
import jax.numpy as jnp
import numpy as np
from jax.experimental import pallas as pl
from jax.experimental.pallas import tpu as pltpu

_DIM = 128
_BITS = 4
_N_LEVELS = 2 ** _BITS


def _lm_levels(bits: int, n_iter: int = 200) -> np.ndarray:
    n = 2 ** bits
    xs = np.linspace(-10.0, 10.0, 400001)
    pdf = np.exp(-np.abs(xs) * math.sqrt(2.0))
    levels = np.linspace(-3.0, 3.0, n)
    for _ in range(n_iter):
        b = (levels[:-1] + levels[1:]) / 2.0
        idx = np.searchsorted(b, xs)
        num = np.bincount(idx, weights=pdf * xs, minlength=n)
        den = np.bincount(idx, weights=pdf, minlength=n)
        levels = np.where(den > 1e-12, num / np.maximum(den, 1e-12), levels)
    return np.sort(levels).astype(np.float32)


_LEVELS = _lm_levels(_BITS)
_BOUNDS = ((_LEVELS[:-1] + _LEVELS[1:]) / 2.0).astype(np.float32)

_BLOCK_ROWS = 8192


def _body(x_ref, o_ref):
    xb = x_ref[...]
    vmin = jnp.min(xb, axis=1, keepdims=True)
    xc = xb - vmin
    ssq = jnp.sum(xc * xc, axis=1, keepdims=True)
    rinv = jax.lax.rsqrt(ssq + 1e-35) * math.sqrt(_DIM)
    xu = xc * rinv
    recon = jnp.full_like(xb, _LEVELS[7])
    for b in range(7, _N_LEVELS - 1):
        recon = jnp.where(xu > _BOUNDS[b], _LEVELS[b + 1], recon)
    num = jnp.sum(xc * recon, axis=1, keepdims=True)
    den = jnp.sum(recon * recon, axis=1, keepdims=True)
    gamma = num / den
    o_ref[...] = recon * gamma + vmin


def kernel(x):
    shape = x.shape
    rows = math.prod(shape[:-1])
    x2 = x.reshape(rows, _DIM)
    grid = rows // _BLOCK_ROWS
    out = pl.pallas_call(
        _body,
        grid=(grid,),
        in_specs=[pl.BlockSpec((_BLOCK_ROWS, _DIM), lambda i: (i, 0))],
        out_specs=pl.BlockSpec((_BLOCK_ROWS, _DIM), lambda i: (i, 0)),
        out_shape=jax.ShapeDtypeStruct((rows, _DIM), jnp.float32),
    )(x2)
    return out.reshape(shape)
